# baseline (speedup 1.0000x reference)
import functools

import jax
import jax.numpy as jnp
from jax import lax
from jax.experimental import pallas as pl
from jax.experimental.pallas import tpu as pltpu
from jax.experimental.pallas import tpu_sc as plsc

VOCAB = 1000
B, T = 1024, 50
NC, NS = 2, 16
NW = NC * NS
NVC, NBC = 8, 8
V0S = (0, 128, 256, 384, 512, 640, 768, 872)
UNITS = T * NVC * NBC
PER_W = UNITS // NW

_mesh = plsc.VectorSubcoreMesh(core_axis_name="c", subcore_axis_name="s")


@functools.partial(
    pl.kernel,
    out_type=jax.ShapeDtypeStruct((T, VOCAB, B), jnp.float32),
    mesh=_mesh,
    scratch_types=[
        pltpu.VMEM((PER_W, 128), jnp.int32),
        pltpu.VMEM((2, 128, 129), jnp.float32),
        pltpu.VMEM((2, 128, 128), jnp.float32),
        pltpu.SemaphoreType.DMA((2,)),
        pltpu.SemaphoreType.DMA((2,)),
    ],
    compiler_params=pltpu.CompilerParams(
        use_tc_tiling_on_sc=True, needs_layout_passes=False),
)
def _sc_gather_t(table_r, idx_u, out_t, idx_v, rows_v, outf_v, gsem, wsem):
    wid = lax.axis_index("s") * NC + lax.axis_index("c")
    pltpu.sync_copy(idx_u.at[wid], idx_v)

    iota = lax.iota(jnp.int32, 16)

    def decode(u):
        unit = wid * PER_W + u
        t = unit // (NVC * NBC)
        r = lax.rem(unit, NVC * NBC)
        vc = r // NBC
        bc = lax.rem(r, NBC)
        return t, vc, bc

    def gather(u, s):
        return pltpu.make_async_copy(
            table_r.at[idx_v.at[u]], rows_v.at[s, :, pl.ds(0, 128)], gsem.at[s])

    def write(u, s):
        t, vc, bc = decode(u)
        v0 = pl.multiple_of(jnp.where(vc == NVC - 1, 872, vc * 128), 8)
        b0 = pl.multiple_of(bc * 128, 128)
        return pltpu.make_async_copy(
            outf_v.at[s],
            out_t.at[t, pl.ds(v0, 128), pl.ds(b0, 128)],
            wsem.at[s])

    def transpose(s):
        sidx = jnp.full((16,), s, jnp.int32)

        @plsc.parallel_loop(0, 128, 1, unroll=8)
        def _(v):
            vfull = jnp.full((16,), v, jnp.int32)
            for bb in range(8):
                vals = plsc.load_gather(
                    rows_v, [sidx, iota + (16 * bb), vfull])
                outf_v[s, v, pl.ds(16 * bb, 16)] = vals

    gather(0, 0).start()
    gather(1, 1).start()

    def body(u, carry):
        s = lax.rem(u, 2)

        @pl.when(u >= 2)
        def _():
            write(u - 2, s).wait()

        gather(u, s).wait()
        transpose(s)

        @pl.when(u + 2 < PER_W)
        def _():
            gather(u + 2, s).start()

        write(u, s).start()
        return carry

    lax.fori_loop(0, PER_W, body, 0)
    write(PER_W - 2, (PER_W - 2) % 2).wait()
    write(PER_W - 1, (PER_W - 1) % 2).wait()


def kernel(idx, table):
    table_r = jnp.stack([lax.slice(table, (0, v0), (VOCAB, v0 + 128))
                         for v0 in V0S])
    table_r = table_r.reshape(NVC * VOCAB, 128)

    idx_t = idx.astype(jnp.int32).T.reshape(T, 1, NBC, 128)
    idx_u = idx_t + (VOCAB * jnp.arange(NVC, dtype=jnp.int32))[None, :, None, None]
    idx_u = idx_u.reshape(NW, PER_W, 128)

    out_t = _sc_gather_t(table_r, idx_u)
    return jnp.transpose(out_t, (2, 0, 1))


# speedup vs baseline: 1.0004x; 1.0004x over previous
import functools

import jax
import jax.numpy as jnp
from jax import lax
from jax.experimental import pallas as pl
from jax.experimental.pallas import tpu as pltpu
from jax.experimental.pallas import tpu_sc as plsc

VOCAB = 1000
B, T = 1024, 50
NC, NS = 2, 16
NW = NC * NS
NVC, NBC = 8, 8
V0S = (0, 128, 256, 384, 512, 640, 768, 872)
UNITS = T * NVC * NBC
PER_W = UNITS // NW

_mesh = plsc.VectorSubcoreMesh(core_axis_name="c", subcore_axis_name="s")


@functools.partial(
    pl.kernel,
    out_type=jax.ShapeDtypeStruct((T, VOCAB, B), jnp.float32),
    mesh=_mesh,
    scratch_types=[
        pltpu.VMEM((PER_W, 128), jnp.int32),
        pltpu.VMEM((2, 128, 129), jnp.float32),
        pltpu.VMEM((2, 128, 128), jnp.float32),
        pltpu.SemaphoreType.DMA((2,)),
        pltpu.SemaphoreType.DMA((2,)),
    ],
    compiler_params=pltpu.CompilerParams(
        use_tc_tiling_on_sc=True, needs_layout_passes=False,
        disable_bounds_checks=True),
)
def _sc_gather_t(table_r, idx_u, out_t, idx_v, rows_v, outf_v, gsem, wsem):
    wid = lax.axis_index("s") * NC + lax.axis_index("c")
    pltpu.sync_copy(idx_u.at[wid], idx_v)

    iota = lax.iota(jnp.int32, 16)

    def decode(u):
        unit = wid * PER_W + u
        t = unit // (NVC * NBC)
        r = lax.rem(unit, NVC * NBC)
        vc = r // NBC
        bc = lax.rem(r, NBC)
        return t, vc, bc

    def gather(u, s):
        return pltpu.make_async_copy(
            table_r.at[idx_v.at[u]], rows_v.at[s, :, pl.ds(0, 128)], gsem.at[s])

    def write(u, s):
        t, vc, bc = decode(u)
        v0 = pl.multiple_of(jnp.where(vc == NVC - 1, 872, vc * 128), 8)
        b0 = pl.multiple_of(bc * 128, 128)
        return pltpu.make_async_copy(
            outf_v.at[s],
            out_t.at[t, pl.ds(v0, 128), pl.ds(b0, 128)],
            wsem.at[s])

    def transpose(s):
        sidx = jnp.full((16,), s, jnp.int32)

        @plsc.parallel_loop(0, 128, 1, unroll=8)
        def _(v):
            vfull = jnp.full((16,), v, jnp.int32)
            for bb in range(8):
                vals = plsc.load_gather(
                    rows_v, [sidx, iota + (16 * bb), vfull])
                outf_v[s, v, pl.ds(16 * bb, 16)] = vals

    gather(0, 0).start()
    gather(1, 1).start()

    def body(u, carry):
        s = lax.rem(u, 2)

        @pl.when(u >= 2)
        def _():
            write(u - 2, s).wait()

        gather(u, s).wait()
        transpose(s)

        @pl.when(u + 2 < PER_W)
        def _():
            gather(u + 2, s).start()

        write(u, s).start()
        return carry

    lax.fori_loop(0, PER_W, body, 0)
    write(PER_W - 2, (PER_W - 2) % 2).wait()
    write(PER_W - 1, (PER_W - 1) % 2).wait()


def kernel(idx, table):
    table_r = jnp.stack([lax.slice(table, (0, v0), (VOCAB, v0 + 128))
                         for v0 in V0S])
    table_r = table_r.reshape(NVC * VOCAB, 128)

    idx_t = idx.astype(jnp.int32).T.reshape(T, 1, NBC, 128)
    idx_u = idx_t + (VOCAB * jnp.arange(NVC, dtype=jnp.int32))[None, :, None, None]
    idx_u = idx_u.reshape(NW, PER_W, 128)

    out_t = _sc_gather_t(table_r, idx_u)
    return jnp.transpose(out_t, (2, 0, 1))


# speedup vs baseline: 3.4361x; 3.4347x over previous
import functools

import jax
import jax.numpy as jnp
from jax import lax
from jax.experimental import pallas as pl
from jax.experimental.pallas import tpu as pltpu
from jax.experimental.pallas import tpu_sc as plsc

VOCAB = 1000
B, T = 1024, 50
NC, NS = 2, 16
NW = NC * NS
NSL = 16
V0S = tuple(64 * i for i in range(15)) + (936,)
BLOCKS = T * 4
LAST_V0 = 936

_mesh = plsc.VectorSubcoreMesh(core_axis_name="c", subcore_axis_name="s")


@functools.partial(
    pl.kernel,
    out_type=jax.ShapeDtypeStruct((T, VOCAB, B), jnp.float32),
    mesh=_mesh,
    scratch_types=[
        pltpu.VMEM((VOCAB // 2, 128), jnp.float32),
        pltpu.VMEM((BLOCKS, 128), jnp.int32),
        pltpu.VMEM((64, 128), jnp.float32),
        pltpu.VMEM((64, 128), jnp.float32),
        pltpu.SemaphoreType.DMA((2,)),
    ],
    compiler_params=pltpu.CompilerParams(
        use_tc_tiling_on_sc=True, needs_layout_passes=False,
        disable_bounds_checks=True),
)
def _sc_gather_t(table_s, idx_u, out_t, tsl, idx_v, outf_a, outf_b, wsem):
    wid = lax.axis_index("s") * NC + lax.axis_index("c")
    c = wid // 2
    h = lax.rem(wid, 2)
    pltpu.sync_copy(table_s.at[c], tsl)
    pltpu.sync_copy(idx_u.at[wid], idx_v)

    iota = lax.iota(jnp.int32, 16)
    outf_s = (outf_a, outf_b)
    v0_abs = pl.multiple_of(jnp.where(c == NSL - 1, LAST_V0, c * 64), 8)

    def write(k, s):
        t = k // 4
        bc = 4 * h + lax.rem(k, 4)
        b0 = pl.multiple_of(bc * 128, 128)
        return pltpu.make_async_copy(
            outf_s[s],
            out_t.at[t, pl.ds(v0_abs, 64), pl.ds(b0, 128)],
            wsem.at[s])

    def transpose(k, s):
        ivecs = [idx_v[k, pl.ds(16 * bb, 16)] for bb in range(8)]
        ihi = [lax.shift_right_logical(iv, 1) for iv in ivecs]
        ilo = [lax.shift_left(iv & 1, 6) for iv in ivecs]
        bvecs = [iota + 16 * bb for bb in range(8)]

        @plsc.parallel_loop(0, 64, 1, unroll=8)
        def _(v):
            vdiag = (jnp.full((16,), v, jnp.int32) - iota) & 63
            for bb in range(8):
                vals = plsc.load_gather(tsl, [ihi[bb], ilo[bb] + vdiag])
                plsc.store_scatter(outf_s[s], [vdiag, bvecs[bb]], vals)

    def body(p, carry):
        for s in range(2):
            k = 2 * p + s

            @pl.when(p >= 1)
            def _():
                write(k - 2, s).wait()

            transpose(k, s)
            write(k, s).start()
        return carry

    lax.fori_loop(0, BLOCKS // 2, body, 0)
    write(BLOCKS - 2, 0).wait()
    write(BLOCKS - 1, 1).wait()


def kernel(idx, table):
    table_s = jnp.stack([lax.slice(table, (0, v0), (VOCAB, v0 + 64))
                         for v0 in V0S])
    table_s = table_s.reshape(NSL, VOCAB // 2, 128)

    idx_t = idx.astype(jnp.int32).T.reshape(T, 2, 4, 128)
    idx_u = jnp.broadcast_to(
        idx_t.transpose(1, 0, 2, 3)[None], (NSL, 2, T, 4, 128))
    idx_u = idx_u.reshape(NW, BLOCKS, 128)

    out_t = _sc_gather_t(table_s, idx_u)
    return jnp.transpose(out_t, (2, 0, 1))
